# baseline (speedup 1.0000x reference)
import functools

import jax
import jax.numpy as jnp
from jax import lax
from jax.experimental import pallas as pl
from jax.experimental.pallas import tpu as pltpu
from jax.experimental.pallas import tpu_sc as plsc

_CHUNKS = (32, 32, 64, 64, 96, 96, 128)
_NCHUNK = len(_CHUNKS)


@functools.lru_cache(maxsize=None)
def _make_gather(V, D, B, NC, NS):
    NW = NC * NS
    b_per_w = B // NW
    assert b_per_w == sum(_CHUNKS) and B % NW == 0
    offs = [sum(_CHUNKS[:i]) for i in range(_NCHUNK)]
    v_main = ((V + NS - 1) // NS + 7) // 8 * 8
    v_last = V - v_main * (NS - 1)
    assert v_last > 0 and v_last % 8 == 0 and V % 8 == 0
    mesh = plsc.VectorSubcoreMesh(core_axis_name="c", subcore_axis_name="s")

    @functools.partial(
        pl.kernel,
        mesh=mesh,
        out_type=jax.ShapeDtypeStruct((B, D), jnp.float32),
        scratch_types=[
            pltpu.VMEM_SHARED((V, D), jnp.float32),
            pltpu.VMEM((b_per_w,), jnp.int32),
            pltpu.VMEM((b_per_w, D), jnp.float32),
            pltpu.SemaphoreType.DMA((2 * _NCHUNK + 2,)),
        ],
    )
    def k(idx_hbm, table_hbm, out_hbm, tab_s, idx_v, rows_v, sems):
        gsem = [sems.at[i] for i in range(_NCHUNK)]
        ssem = [sems.at[_NCHUNK + i] for i in range(_NCHUNK)]
        tsem, isem = sems.at[2 * _NCHUNK], sems.at[2 * _NCHUNK + 1]
        cid = lax.axis_index("c")
        sid = lax.axis_index("s")
        wid = sid * NC + cid
        base = wid * b_per_w
        icopy = pltpu.async_copy(idx_hbm.at[pl.ds(base, b_per_w)], idx_v, isem)

        @pl.when(sid != NS - 1)
        def _():
            pltpu.async_copy(
                table_hbm.at[pl.ds(sid * v_main, v_main)],
                tab_s.at[pl.ds(sid * v_main, v_main)],
                tsem,
            ).wait()

        @pl.when(sid == NS - 1)
        def _():
            pltpu.async_copy(
                table_hbm.at[pl.ds((NS - 1) * v_main, v_last)],
                tab_s.at[pl.ds((NS - 1) * v_main, v_last)],
                tsem,
            ).wait()

        icopy.wait()
        plsc.subcore_barrier()
        gathers = [
            pltpu.async_copy(
                tab_s.at[idx_v.at[pl.ds(offs[i], _CHUNKS[i])]],
                rows_v.at[pl.ds(offs[i], _CHUNKS[i])],
                gsem[i],
            )
            for i in range(_NCHUNK)
        ]
        stores = []
        for i in range(_NCHUNK):
            gathers[i].wait()
            stores.append(
                pltpu.async_copy(
                    rows_v.at[pl.ds(offs[i], _CHUNKS[i])],
                    out_hbm.at[pl.ds(base + offs[i], _CHUNKS[i])],
                    ssem[i],
                )
            )
        for s in stores:
            s.wait()

    return k


def kernel(t, time_encodings):
    t = t.astype(jnp.int32)
    (B,) = t.shape
    V, D = time_encodings.shape
    info = plsc.get_sparse_core_info()
    k = _make_gather(V, D, B, info.num_cores, info.num_subcores)
    return k(t, time_encodings)


# speedup vs baseline: 1.0034x; 1.0034x over previous
import functools

import jax
import jax.numpy as jnp
from jax import lax
from jax.experimental import pallas as pl
from jax.experimental.pallas import tpu as pltpu
from jax.experimental.pallas import tpu_sc as plsc

_NCHUNK = 8


@functools.lru_cache(maxsize=None)
def _make_gather(V, D, B, NC, NS):
    NW = NC * NS
    b_per_w = B // NW
    assert B % (NW * _NCHUNK) == 0
    _CHUNKS = (b_per_w // _NCHUNK,) * _NCHUNK
    offs = [sum(_CHUNKS[:i]) for i in range(_NCHUNK)]
    v_main = ((V + NS - 1) // NS + 7) // 8 * 8
    assert v_main % 8 == 0 and (V - v_main) % 8 == 0 and v_main * NS >= V
    mesh = plsc.VectorSubcoreMesh(core_axis_name="c", subcore_axis_name="s")

    @functools.partial(
        pl.kernel,
        mesh=mesh,
        out_type=jax.ShapeDtypeStruct((B, D), jnp.float32),
        scratch_types=[
            pltpu.VMEM_SHARED((V, D), jnp.float32),
            pltpu.VMEM((b_per_w,), jnp.int32),
            pltpu.VMEM((b_per_w, D), jnp.float32),
            pltpu.SemaphoreType.DMA((2 * _NCHUNK + 2,)),
        ],
    )
    def k(idx_hbm, table_hbm, out_hbm, tab_s, idx_v, rows_v, sems):
        gsem = [sems.at[i] for i in range(_NCHUNK)]
        ssem = [sems.at[_NCHUNK + i] for i in range(_NCHUNK)]
        tsem, isem = sems.at[2 * _NCHUNK], sems.at[2 * _NCHUNK + 1]
        cid = lax.axis_index("c")
        sid = lax.axis_index("s")
        wid = sid * NC + cid
        base = wid * b_per_w
        icopy = pltpu.async_copy(idx_hbm.at[pl.ds(base, b_per_w)], idx_v, isem)
        voff = lax.min(sid * v_main, V - v_main)
        pltpu.async_copy(
            table_hbm.at[pl.ds(voff, v_main)],
            tab_s.at[pl.ds(voff, v_main)],
            tsem,
        ).wait()
        icopy.wait()
        plsc.subcore_barrier()
        gathers = [
            pltpu.async_copy(
                tab_s.at[idx_v.at[pl.ds(offs[i], _CHUNKS[i])]],
                rows_v.at[pl.ds(offs[i], _CHUNKS[i])],
                gsem[i],
            )
            for i in range(_NCHUNK)
        ]
        stores = []
        for i in range(_NCHUNK):
            gathers[i].wait()
            stores.append(
                pltpu.async_copy(
                    rows_v.at[pl.ds(offs[i], _CHUNKS[i])],
                    out_hbm.at[pl.ds(base + offs[i], _CHUNKS[i])],
                    ssem[i],
                )
            )
        for s in stores:
            s.wait()

    return k


def kernel(t, time_encodings):
    t = t.astype(jnp.int32)
    (B,) = t.shape
    V, D = time_encodings.shape
    info = plsc.get_sparse_core_info()
    k = _make_gather(V, D, B, info.num_cores, info.num_subcores)
    return k(t, time_encodings)
